# Initial kernel scaffold; baseline (speedup 1.0000x reference)
import jax
import jax.numpy as jnp
from jax.experimental import pallas as pl


def kernel(y):
    raise NotImplementedError("write your pallas kernel here")


# speedup vs baseline: 2.9940x; 2.9940x over previous
import jax
import jax.numpy as jnp
from jax import lax
from jax.experimental import pallas as pl
from jax.experimental.pallas import tpu as pltpu
from jax.experimental.pallas import tpu_sc as plsc

_T = 8192
_F = 4096
_M = 2 * _T - 1
_NC = 2
_NS = 16
_NW = _NC * _NS
_TW = _T // _NW
_K = 8
_NCHUNK = _TW // _K
_L = 16
_NJ = _F // _L


def _sc_body(y_hbm, out_hbm, vbuf, obuf):
    wid = lax.axis_index("s") * _NC + lax.axis_index("c")
    base = wid * _TW

    def chunk_body(ci, carry):
        m0 = base + ci * _K
        start = jnp.clip(m0 - 1, 0, _T - (_K + 2))
        pltpu.sync_copy(y_hbm.at[pl.ds(start * _F, (_K + 2) * _F)], vbuf)

        def m_body(i, carry):
            m = m0 + i
            li = m - start
            pli = jnp.maximum(li - 1, 0)
            nli = jnp.minimum(li + 1, _K + 1)
            mf = m.astype(jnp.float32)
            a = mf * (1.0 / _M)
            ca = 1.0 - a
            b = (mf + _T) * (1.0 / _M)
            cb = 1.0 - b

            def j_body(j, carry):
                off = j * _L
                pv = vbuf[pl.ds(pli * _F + off, _L)]
                cv = vbuf[pl.ds(li * _F + off, _L)]
                nv = vbuf[pl.ds(nli * _F + off, _L)]
                obuf[pl.ds(2 * i * _F + off, _L)] = a * pv + ca * cv
                obuf[pl.ds((2 * i + 1) * _F + off, _L)] = b * cv + cb * nv
                return carry

            return lax.fori_loop(0, _NJ, j_body, carry)

        lax.fori_loop(0, _K, m_body, 0)
        pltpu.sync_copy(obuf, out_hbm.at[pl.ds(2 * m0 * _F, 2 * _K * _F)])
        return carry

    lax.fori_loop(0, _NCHUNK, chunk_body, 0)


def kernel(y):
    y1 = y.reshape(_T * _F)
    mesh = plsc.VectorSubcoreMesh(core_axis_name="c", subcore_axis_name="s")
    out = pl.kernel(
        _sc_body,
        mesh=mesh,
        out_type=jax.ShapeDtypeStruct((2 * _T * _F,), jnp.float32),
        scratch_types=[
            pltpu.VMEM(((_K + 2) * _F,), jnp.float32),
            pltpu.VMEM((2 * _K * _F,), jnp.float32),
        ],
    )(y1)
    return out.reshape(2 * _T, 4, 1024)


# speedup vs baseline: 5.3671x; 1.7926x over previous
import jax
import jax.numpy as jnp
from jax import lax
from jax.experimental import pallas as pl
from jax.experimental.pallas import tpu as pltpu
from jax.experimental.pallas import tpu_sc as plsc

_T = 8192
_F = 4096
_M = 2 * _T - 1
_NC = 2
_NS = 16
_NW = _NC * _NS
_TW = _T // _NW
_K = 4
_NCHUNK = _TW // _K
_L = 16
_NPAIR = _NCHUNK // 2


def _sc_body(y_hbm, out_hbm, vb0, vb1, ob0, ob1, ls0, ls1, ss0, ss1):
    wid = lax.axis_index("s") * _NC + lax.axis_index("c")
    base = wid * _TW
    vbufs = (vb0, vb1)
    obufs = (ob0, ob1)
    lsems = (ls0, ls1)
    ssems = (ss0, ss1)

    def issue_load(ci, b):
        m0 = base + ci * _K
        prow = jnp.maximum(m0 - 1, 0)
        nrow = jnp.minimum(m0 + _K, _T - 1)
        pltpu.async_copy(y_hbm.at[pl.ds(prow * _F, _F)],
                         vbufs[b].at[pl.ds(0, _F)], lsems[b])
        pltpu.async_copy(y_hbm.at[pl.ds(m0 * _F, _K * _F)],
                         vbufs[b].at[pl.ds(_F, _K * _F)], lsems[b])
        pltpu.async_copy(y_hbm.at[pl.ds(nrow * _F, _F)],
                         vbufs[b].at[pl.ds((_K + 1) * _F, _F)], lsems[b])

    def wait_load(b):
        pltpu.make_async_copy(y_hbm.at[pl.ds(0, (_K + 2) * _F)],
                              vbufs[b], lsems[b]).wait()

    def issue_store(ci, b):
        m0 = base + ci * _K
        pltpu.async_copy(obufs[b],
                         out_hbm.at[pl.ds(2 * m0 * _F, 2 * _K * _F)], ssems[b])

    def wait_store(b):
        pltpu.make_async_copy(obufs[b],
                              out_hbm.at[pl.ds(0, 2 * _K * _F)], ssems[b]).wait()

    def compute(ci, b):
        m0f = (base + ci * _K).astype(jnp.float32)
        avs = []
        bvs = []
        for i in range(_K):
            a = (m0f + i) * (1.0 / _M)
            bw = (m0f + (i + _T)) * (1.0 / _M)
            avs.append(jnp.broadcast_to(a, (_L,)))
            bvs.append(jnp.broadcast_to(bw, (_L,)))
        vb = vbufs[b]
        ob = obufs[b]

        @plsc.parallel_loop(0, _F, _L, unroll=2)
        def _(j):
            lv = [vb[pl.ds(r * _F + j, _L)] for r in range(_K + 2)]
            diff = [lv[r] - lv[r + 1] for r in range(_K + 1)]
            for i in range(_K):
                ob[pl.ds((2 * i) * _F + j, _L)] = lv[i + 1] + avs[i] * diff[i]
                ob[pl.ds((2 * i + 1) * _F + j, _L)] = lv[i + 2] + bvs[i] * diff[i + 1]

    issue_load(0, 0)
    issue_load(1, 1)

    def pair_body(g, carry):
        for b in range(2):
            ci = 2 * g + b
            wait_load(b)

            @pl.when(g >= 1)
            def _():
                wait_store(b)

            compute(ci, b)
            issue_store(ci, b)

            @pl.when(g <= _NPAIR - 2)
            def _():
                issue_load(ci + 2, b)

        return carry

    lax.fori_loop(0, _NPAIR, pair_body, 0)
    wait_store(0)
    wait_store(1)


def kernel(y):
    y1 = y.reshape(_T * _F)
    mesh = plsc.VectorSubcoreMesh(core_axis_name="c", subcore_axis_name="s")
    out = pl.kernel(
        _sc_body,
        mesh=mesh,
        out_type=jax.ShapeDtypeStruct((2 * _T * _F,), jnp.float32),
        scratch_types=[
            pltpu.VMEM(((_K + 2) * _F,), jnp.float32),
            pltpu.VMEM(((_K + 2) * _F,), jnp.float32),
            pltpu.VMEM((2 * _K * _F,), jnp.float32),
            pltpu.VMEM((2 * _K * _F,), jnp.float32),
            pltpu.SemaphoreType.DMA,
            pltpu.SemaphoreType.DMA,
            pltpu.SemaphoreType.DMA,
            pltpu.SemaphoreType.DMA,
        ],
    )(y1)
    return out.reshape(2 * _T, 4, 1024)
